# baseline (speedup 1.0000x reference)
import functools

import jax
import jax.numpy as jnp
from jax import lax
from jax.experimental import pallas as pl
from jax.experimental.pallas import tpu as pltpu
from jax.experimental.pallas import tpu_sc as plsc

N = 10000
E = 160000
HID = 32

NC = 2
NS = 16
NW = NC * NS
CB = 128
CH = 40
EPT = CH * CB
E_PAD = NW * EPT
N_PAD = 10240
RPS = N_PAD // NS

_f32 = jnp.float32


def _mesh():
    return plsc.VectorSubcoreMesh(core_axis_name="c", subcore_axis_name="s")


KCH = 2
NBLK = CH // KCH


def _build_gather():
    KG = 2
    NBG = CH // KG

    @functools.partial(
        pl.kernel,
        out_type=jax.ShapeDtypeStruct((E_PAD, 128), _f32),
        mesh=_mesh(),
        scratch_types=[
            pltpu.VMEM((CH, CB), jnp.int32),
            pltpu.VMEM((KG * CB, 128), _f32),
            pltpu.SemaphoreType.DMA,
        ],
    )
    def gather(table_hbm, src_hbm, out_hbm, idx_v, rows_v, sem):
        c = lax.axis_index("c")
        s = lax.axis_index("s")
        w = c * NS + s
        pltpu.sync_copy(src_hbm.at[w], idx_v)

        def body(b, carry):
            descs = [
                pltpu.async_copy(table_hbm.at[idx_v.at[b * KG + t]],
                                 rows_v.at[pl.ds(t * CB, CB)], sem)
                for t in range(KG)
            ]
            for d in descs:
                d.wait()
            pltpu.sync_copy(
                rows_v, out_hbm.at[pl.ds(w * EPT + b * (KG * CB), KG * CB)])
            return carry

        lax.fori_loop(0, NBG, body, 0)

    return gather


def _build_counts():
    @functools.partial(
        pl.kernel,
        out_type=jax.ShapeDtypeStruct((NC, N_PAD, HID), _f32),
        mesh=_mesh(),
        scratch_types=[
            pltpu.VMEM((CH, CB), jnp.int32),
            pltpu.VMEM((CB, HID), _f32),
            pltpu.VMEM_SHARED((N_PAD, HID), _f32),
            pltpu.SemaphoreType.DMA,
        ],
    )
    def counts(dst_hbm, zeros_hbm, ones_hbm, cnt_out, idx_v, ones_v, cnt_sh,
               sem):
        c = lax.axis_index("c")
        s = lax.axis_index("s")
        w = c * NS + s
        pltpu.sync_copy(dst_hbm.at[w], idx_v)
        pltpu.sync_copy(ones_hbm, ones_v)
        pltpu.sync_copy(zeros_hbm.at[pl.ds(s * RPS, RPS)],
                        cnt_sh.at[pl.ds(s * RPS, RPS)])
        plsc.subcore_barrier()

        def body(b, carry):
            descs = [
                pltpu.async_copy(ones_v, cnt_sh.at[idx_v.at[b * KCH + t]],
                                 sem, add=True)
                for t in range(KCH)
            ]
            for d in descs:
                d.wait()
            return carry

        lax.fori_loop(0, NBLK, body, 0)
        plsc.subcore_barrier()
        pltpu.sync_copy(cnt_sh.at[pl.ds(s * RPS, RPS)],
                        cnt_out.at[c, pl.ds(s * RPS, RPS)])

    return counts


def _build_scatter():
    @functools.partial(
        pl.kernel,
        out_type=jax.ShapeDtypeStruct((NC, N_PAD, HID), _f32),
        mesh=_mesh(),
        scratch_types=[
            pltpu.VMEM((CH, CB), jnp.int32),
            pltpu.VMEM((KCH * CB, HID), _f32),
            pltpu.VMEM_SHARED((N_PAD, HID), _f32),
            pltpu.SemaphoreType.DMA,
        ],
    )
    def scatter(msg_hbm, dst_hbm, zeros_hbm, agg_out, idx_v, msg_v, agg_sh,
                sem):
        c = lax.axis_index("c")
        s = lax.axis_index("s")
        w = c * NS + s
        pltpu.sync_copy(dst_hbm.at[w], idx_v)
        pltpu.sync_copy(zeros_hbm.at[pl.ds(s * RPS, RPS)],
                        agg_sh.at[pl.ds(s * RPS, RPS)])
        plsc.subcore_barrier()

        def body(b, carry):
            pltpu.sync_copy(
                msg_hbm.at[pl.ds(w * EPT + b * (KCH * CB), KCH * CB)], msg_v)
            descs = [
                pltpu.async_copy(msg_v.at[pl.ds(t * CB, CB)],
                                 agg_sh.at[idx_v.at[b * KCH + t]], sem,
                                 add=True)
                for t in range(KCH)
            ]
            for d in descs:
                d.wait()
            return carry

        lax.fori_loop(0, NBLK, body, 0)
        plsc.subcore_barrier()
        pltpu.sync_copy(agg_sh.at[pl.ds(s * RPS, RPS)],
                        agg_out.at[c, pl.ds(s * RPS, RPS)])

    return scatter


_sc_gather = _build_gather()
_sc_scatter = _build_scatter()
_sc_counts = _build_counts()


_BN1 = 1024
_TE2 = 2048
_TE = 1024
_BN5 = 512


def _h0_body(x_ref, t_ref, wt1_ref, bt1_ref, wt2_ref, bt2_ref, out_ref):
    t1 = jnp.maximum(
        jnp.dot(t_ref[...], wt1_ref[...], preferred_element_type=_f32)
        + bt1_ref[...], 0.0)
    t2 = jnp.maximum(
        jnp.dot(t1, wt2_ref[...], preferred_element_type=_f32)
        + bt2_ref[...], 0.0)
    pad = jnp.zeros((x_ref.shape[0], 128 - HID), _f32)
    out_ref[...] = jnp.concatenate([x_ref[...], t2, pad], axis=1)


def _msg_body(ea_ref, we1_ref, be1_ref, h_ref, vcat_ref, esel_ref, bm_ref,
              out_ref):
    z = jnp.maximum(
        jnp.dot(ea_ref[...], we1_ref[...], preferred_element_type=_f32)
        + be1_ref[...], 0.0).astype(jnp.bfloat16)
    h = h_ref[:, :HID].astype(jnp.bfloat16)
    g2 = jnp.dot(h, vcat_ref[...],
                 preferred_element_type=_f32).astype(jnp.bfloat16)
    z128 = jnp.concatenate([z, z, z, z], axis=1)
    z1024 = jnp.concatenate([z128] * 8, axis=1)
    out_ref[...] = (
        jnp.dot(z1024 * g2, esel_ref[...], preferred_element_type=_f32)
        + jnp.dot(h, bm_ref[...], preferred_element_type=_f32))


def _combine_body(a0_ref, a1_ref, c0_ref, c1_ref, h_ref, root_ref, bias_ref,
                  out_ref):
    denom = jnp.maximum(c0_ref[...] + c1_ref[...], 1.0)
    agg = (a0_ref[...] + a1_ref[...]) / denom
    h1 = jnp.maximum(
        agg + jnp.dot(h_ref[:, :HID], root_ref[...],
                      preferred_element_type=_f32)
        + bias_ref[...], 0.0)
    pad = jnp.zeros((h1.shape[0], 128 - HID), _f32)
    out_ref[...] = jnp.concatenate([h1, pad], axis=1)


def _combine_head_body(a0_ref, a1_ref, c0_ref, c1_ref, h_ref, root_ref,
                       bias_ref, wout_ref, bout_ref, out_ref):
    denom = jnp.maximum(c0_ref[...] + c1_ref[...], 1.0)
    agg = (a0_ref[...] + a1_ref[...]) / denom
    h2 = jnp.maximum(
        agg + jnp.dot(h_ref[:, :HID], root_ref[...],
                      preferred_element_type=_f32)
        + bias_ref[...], 0.0)
    logits = jnp.dot(h2, wout_ref[...], preferred_element_type=_f32) \
        + bout_ref[...]
    out_ref[...] = jax.nn.sigmoid(logits)


def _full(shape):
    return pl.BlockSpec(shape, lambda i: (0,) * len(shape))


def _rows(bn, w):
    return pl.BlockSpec((bn, w), lambda i: (i, 0))


def kernel(x, topo, edge_attr, edge_index, Wt1, bt1, Wt2, bt2, We1, be1,
           We2, be2, root1, bias1, root2, bias2, Wout, bout):
    node_in = x.shape[1]
    topo_in = topo.shape[1]
    tproj = Wt1.shape[1]
    edge_in = edge_attr.shape[1]

    xp = jnp.pad(x, ((0, N_PAD - N), (0, 0)))
    tp = jnp.pad(topo, ((0, N_PAD - N), (0, 0)))
    eap = jnp.pad(edge_attr, ((0, E_PAD - E), (0, 0)))
    src3 = jnp.pad(edge_index[0], (0, E_PAD - E)).reshape(NW, CH, CB)
    dst3 = jnp.pad(edge_index[1], (0, E_PAD - E),
                   constant_values=N).reshape(NW, CH, CB)
    Vcat2 = (We2.reshape(HID, HID, HID).transpose(1, 2, 0)
             .reshape(HID, HID * HID).astype(jnp.bfloat16))
    Esel = jnp.kron(jnp.eye(HID, dtype=_f32),
                    jnp.ones((HID, 1), _f32)).astype(jnp.bfloat16)
    Bm = be2.reshape(HID, HID)
    zeros_tab = jnp.zeros((N_PAD, HID), _f32)
    ones_blk = jnp.ones((CB, HID), _f32)
    bt1r = bt1.reshape(1, tproj)
    bt2r = bt2.reshape(1, tproj)
    be1r = be1.reshape(1, HID)
    b1r = bias1.reshape(1, HID)
    b2r = bias2.reshape(1, HID)
    boutr = bout.reshape(1, 1)

    h0 = pl.pallas_call(
        _h0_body,
        grid=(N_PAD // _BN1,),
        in_specs=[_rows(_BN1, node_in), _rows(_BN1, topo_in),
                  _full((topo_in, tproj)), _full((1, tproj)),
                  _full((tproj, tproj)), _full((1, tproj))],
        out_specs=_rows(_BN1, 128),
        out_shape=jax.ShapeDtypeStruct((N_PAD, 128), _f32),
    )(xp, tp, Wt1, bt1r, Wt2, bt2r)

    def msg_call(hsrc):
        return pl.pallas_call(
            _msg_body,
            grid=(E_PAD // _TE,),
            in_specs=[_rows(_TE, edge_in), _full((edge_in, HID)),
                      _full((1, HID)), _rows(_TE, 128),
                      _full((HID, HID * HID)), _full((HID * HID, HID)),
                      _full((HID, HID))],
            out_specs=_rows(_TE, HID),
            out_shape=jax.ShapeDtypeStruct((E_PAD, HID), _f32),
        )(eap, We1, be1r, hsrc, Vcat2, Esel, Bm)

    hsrc1 = _sc_gather(h0, src3)
    msg1 = msg_call(hsrc1)
    aggp1 = _sc_scatter(msg1, dst3, zeros_tab)
    cntp = _sc_counts(dst3, zeros_tab, ones_blk)

    h1 = pl.pallas_call(
        _combine_body,
        grid=(N_PAD // _BN5,),
        in_specs=[_rows(_BN5, HID)] * 4
        + [_rows(_BN5, 128), _full((HID, HID)), _full((1, HID))],
        out_specs=_rows(_BN5, 128),
        out_shape=jax.ShapeDtypeStruct((N_PAD, 128), _f32),
    )(aggp1[0], aggp1[1], cntp[0], cntp[1], h0, root1, b1r)

    hsrc2 = _sc_gather(h1, src3)
    msg2 = msg_call(hsrc2)
    aggp2 = _sc_scatter(msg2, dst3, zeros_tab)

    out = pl.pallas_call(
        _combine_head_body,
        grid=(N_PAD // _BN5,),
        in_specs=[_rows(_BN5, HID)] * 4
        + [_rows(_BN5, 128), _full((HID, HID)), _full((1, HID)),
           _full((HID, 1)), _full((1, 1))],
        out_specs=_rows(_BN5, 1),
        out_shape=jax.ShapeDtypeStruct((N_PAD, 1), _f32),
    )(aggp2[0], aggp2[1], cntp[0], cntp[1], h1, root2, b2r, Wout, boutr)

    return out[:N]


# speedup vs baseline: 1.0271x; 1.0271x over previous
import functools

import jax
import jax.numpy as jnp
from jax import lax
from jax.experimental import pallas as pl
from jax.experimental.pallas import tpu as pltpu
from jax.experimental.pallas import tpu_sc as plsc

N = 10000
E = 160000
HID = 32

NC = 2
NS = 16
NW = NC * NS
CB = 128
CH = 40
EPT = CH * CB
E_PAD = NW * EPT
N_PAD = 10240
RPS = N_PAD // NS

_f32 = jnp.float32


def _mesh():
    return plsc.VectorSubcoreMesh(core_axis_name="c", subcore_axis_name="s")


KCH = 2
NBLK = CH // KCH


def _build_gather():
    KG = 2
    NBG = CH // KG

    @functools.partial(
        pl.kernel,
        out_type=jax.ShapeDtypeStruct((E_PAD, 128), _f32),
        mesh=_mesh(),
        scratch_types=[
            pltpu.VMEM((CH, CB), jnp.int32),
            pltpu.VMEM((KG * CB, 128), _f32),
            pltpu.SemaphoreType.DMA,
        ],
    )
    def gather(table_hbm, src_hbm, out_hbm, idx_v, rows_v, sem):
        c = lax.axis_index("c")
        s = lax.axis_index("s")
        w = c * NS + s
        pltpu.sync_copy(src_hbm.at[w], idx_v)

        def body(b, carry):
            descs = [
                pltpu.async_copy(table_hbm.at[idx_v.at[b * KG + t]],
                                 rows_v.at[pl.ds(t * CB, CB)], sem)
                for t in range(KG)
            ]
            for d in descs:
                d.wait()
            pltpu.sync_copy(
                rows_v, out_hbm.at[pl.ds(w * EPT + b * (KG * CB), KG * CB)])
            return carry

        lax.fori_loop(0, NBG, body, 0)

    return gather


def _build_counts():
    @functools.partial(
        pl.kernel,
        out_type=jax.ShapeDtypeStruct((NC, N_PAD, HID), _f32),
        mesh=_mesh(),
        scratch_types=[
            pltpu.VMEM((CH, CB), jnp.int32),
            pltpu.VMEM((CB, HID), _f32),
            pltpu.VMEM_SHARED((N_PAD, HID), _f32),
            pltpu.SemaphoreType.DMA,
        ],
    )
    def counts(dst_hbm, zeros_hbm, ones_hbm, cnt_out, idx_v, ones_v, cnt_sh,
               sem):
        c = lax.axis_index("c")
        s = lax.axis_index("s")
        w = c * NS + s
        pltpu.sync_copy(dst_hbm.at[w], idx_v)
        pltpu.sync_copy(ones_hbm, ones_v)
        pltpu.sync_copy(zeros_hbm.at[pl.ds(s * RPS, RPS)],
                        cnt_sh.at[pl.ds(s * RPS, RPS)])
        plsc.subcore_barrier()

        def body(b, carry):
            descs = [
                pltpu.async_copy(ones_v, cnt_sh.at[idx_v.at[b * KCH + t]],
                                 sem, add=True)
                for t in range(KCH)
            ]
            for d in descs:
                d.wait()
            return carry

        lax.fori_loop(0, NBLK, body, 0)
        plsc.subcore_barrier()
        pltpu.sync_copy(cnt_sh.at[pl.ds(s * RPS, RPS)],
                        cnt_out.at[c, pl.ds(s * RPS, RPS)])

    return counts


def _build_scatter(W):
    @functools.partial(
        pl.kernel,
        out_type=jax.ShapeDtypeStruct((NC, N_PAD, W), _f32),
        mesh=_mesh(),
        scratch_types=[
            pltpu.VMEM((CH, CB), jnp.int32),
            pltpu.VMEM((KCH * CB, W), _f32),
            pltpu.VMEM_SHARED((N_PAD, W), _f32),
            pltpu.SemaphoreType.DMA,
        ],
    )
    def scatter(msg_hbm, dst_hbm, zeros_hbm, agg_out, idx_v, msg_v, agg_sh,
                sem):
        c = lax.axis_index("c")
        s = lax.axis_index("s")
        w = c * NS + s
        pltpu.sync_copy(dst_hbm.at[w], idx_v)
        pltpu.sync_copy(zeros_hbm.at[pl.ds(s * RPS, RPS)],
                        agg_sh.at[pl.ds(s * RPS, RPS)])
        plsc.subcore_barrier()

        def body(b, carry):
            pltpu.sync_copy(
                msg_hbm.at[pl.ds(w * EPT + b * (KCH * CB), KCH * CB)], msg_v)
            descs = [
                pltpu.async_copy(msg_v.at[pl.ds(t * CB, CB)],
                                 agg_sh.at[idx_v.at[b * KCH + t]], sem,
                                 add=True)
                for t in range(KCH)
            ]
            for d in descs:
                d.wait()
            return carry

        lax.fori_loop(0, NBLK, body, 0)
        plsc.subcore_barrier()
        pltpu.sync_copy(agg_sh.at[pl.ds(s * RPS, RPS)],
                        agg_out.at[c, pl.ds(s * RPS, RPS)])

    return scatter


_sc_gather = _build_gather()
_sc_scatter64 = _build_scatter(2 * HID)
_sc_scatter32 = _build_scatter(HID)


_BN1 = 1024
_TE2 = 2048
_TE = 1024
_BN5 = 512


def _h0_body(x_ref, t_ref, wt1_ref, bt1_ref, wt2_ref, bt2_ref, out_ref):
    t1 = jnp.maximum(
        jnp.dot(t_ref[...], wt1_ref[...], preferred_element_type=_f32)
        + bt1_ref[...], 0.0)
    t2 = jnp.maximum(
        jnp.dot(t1, wt2_ref[...], preferred_element_type=_f32)
        + bt2_ref[...], 0.0)
    pad = jnp.zeros((x_ref.shape[0], 128 - HID), _f32)
    out_ref[...] = jnp.concatenate([x_ref[...], t2, pad], axis=1)


def _z_body(ea_ref, we1_ref, be1_ref, out_ref):
    out_ref[...] = jnp.maximum(
        jnp.dot(ea_ref[...], we1_ref[...], preferred_element_type=_f32)
        + be1_ref[...], 0.0).astype(jnp.bfloat16)


def _msg_body(z_ref, h_ref, vcat_ref, esel_ref, bm_ref, out_ref):
    z = z_ref[...]
    h = h_ref[:, :HID].astype(jnp.bfloat16)
    g2 = jnp.dot(h, vcat_ref[...],
                 preferred_element_type=_f32).astype(jnp.bfloat16)
    z128 = jnp.concatenate([z, z, z, z], axis=1)
    z1024 = jnp.concatenate([z128] * 8, axis=1)
    out_ref[...] = (
        jnp.dot(z1024 * g2, esel_ref[...], preferred_element_type=_f32)
        + jnp.dot(h, bm_ref[...], preferred_element_type=_f32))


def _msg64_body(z_ref, h_ref, vcat_ref, esel_ref, bm_ref, out_ref):
    z = z_ref[...]
    h = h_ref[:, :HID].astype(jnp.bfloat16)
    g2 = jnp.dot(h, vcat_ref[...],
                 preferred_element_type=_f32).astype(jnp.bfloat16)
    z128 = jnp.concatenate([z, z, z, z], axis=1)
    z1024 = jnp.concatenate([z128] * 8, axis=1)
    m = (jnp.dot(z1024 * g2, esel_ref[...], preferred_element_type=_f32)
         + jnp.dot(h, bm_ref[...], preferred_element_type=_f32))
    out_ref[...] = jnp.concatenate(
        [m, jnp.ones((m.shape[0], HID), _f32)], axis=1)


def _combine_body(a0_ref, a1_ref, h_ref, root_ref, bias_ref, out_ref):
    a0 = a0_ref[...]
    a1 = a1_ref[...]
    denom = jnp.maximum(a0[:, HID:] + a1[:, HID:], 1.0)
    agg = (a0[:, :HID] + a1[:, :HID]) / denom
    h1 = jnp.maximum(
        agg + jnp.dot(h_ref[:, :HID], root_ref[...],
                      preferred_element_type=_f32)
        + bias_ref[...], 0.0)
    pad = jnp.zeros((h1.shape[0], 128 - HID), _f32)
    out_ref[...] = jnp.concatenate([h1, pad], axis=1)


def _combine_head_body(a0_ref, a1_ref, c0_ref, c1_ref, h_ref, root_ref,
                       bias_ref, wout_ref, bout_ref, out_ref):
    denom = jnp.maximum(c0_ref[:, HID:] + c1_ref[:, HID:], 1.0)
    agg = (a0_ref[...] + a1_ref[...]) / denom
    h2 = jnp.maximum(
        agg + jnp.dot(h_ref[:, :HID], root_ref[...],
                      preferred_element_type=_f32)
        + bias_ref[...], 0.0)
    logits = jnp.dot(h2, wout_ref[...], preferred_element_type=_f32) \
        + bout_ref[...]
    out_ref[...] = jax.nn.sigmoid(logits)


def _full(shape):
    return pl.BlockSpec(shape, lambda i: (0,) * len(shape))


def _rows(bn, w):
    return pl.BlockSpec((bn, w), lambda i: (i, 0))


def kernel(x, topo, edge_attr, edge_index, Wt1, bt1, Wt2, bt2, We1, be1,
           We2, be2, root1, bias1, root2, bias2, Wout, bout):
    node_in = x.shape[1]
    topo_in = topo.shape[1]
    tproj = Wt1.shape[1]
    edge_in = edge_attr.shape[1]

    xp = jnp.pad(x, ((0, N_PAD - N), (0, 0)))
    tp = jnp.pad(topo, ((0, N_PAD - N), (0, 0)))
    eap = jnp.pad(edge_attr, ((0, E_PAD - E), (0, 0)))
    src3 = jnp.pad(edge_index[0], (0, E_PAD - E)).reshape(NW, CH, CB)
    dst3 = jnp.pad(edge_index[1], (0, E_PAD - E),
                   constant_values=N).reshape(NW, CH, CB)
    Vcat2 = (We2.reshape(HID, HID, HID).transpose(1, 2, 0)
             .reshape(HID, HID * HID).astype(jnp.bfloat16))
    Esel = jnp.kron(jnp.eye(HID, dtype=_f32),
                    jnp.ones((HID, 1), _f32)).astype(jnp.bfloat16)
    Bm = be2.reshape(HID, HID)
    zeros_tab = jnp.zeros((N_PAD, HID), _f32)
    zeros_tab64 = jnp.zeros((N_PAD, 2 * HID), _f32)
    bt1r = bt1.reshape(1, tproj)
    bt2r = bt2.reshape(1, tproj)
    be1r = be1.reshape(1, HID)
    b1r = bias1.reshape(1, HID)
    b2r = bias2.reshape(1, HID)
    boutr = bout.reshape(1, 1)

    h0 = pl.pallas_call(
        _h0_body,
        grid=(N_PAD // _BN1,),
        in_specs=[_rows(_BN1, node_in), _rows(_BN1, topo_in),
                  _full((topo_in, tproj)), _full((1, tproj)),
                  _full((tproj, tproj)), _full((1, tproj))],
        out_specs=_rows(_BN1, 128),
        out_shape=jax.ShapeDtypeStruct((N_PAD, 128), _f32),
    )(xp, tp, Wt1, bt1r, Wt2, bt2r)

    def msg_call(hsrc):
        return pl.pallas_call(
            _msg_body,
            grid=(E_PAD // _TE,),
            in_specs=[_rows(_TE, HID), _rows(_TE, 128),
                      _full((HID, HID * HID)), _full((HID * HID, HID)),
                      _full((HID, HID))],
            out_specs=_rows(_TE, HID),
            out_shape=jax.ShapeDtypeStruct((E_PAD, HID), _f32),
        )(z, hsrc, Vcat2, Esel, Bm)

    z = pl.pallas_call(
        _z_body,
        grid=(E_PAD // _TE2,),
        in_specs=[_rows(_TE2, edge_in), _full((edge_in, HID)),
                  _full((1, HID))],
        out_specs=_rows(_TE2, HID),
        out_shape=jax.ShapeDtypeStruct((E_PAD, HID), jnp.bfloat16),
    )(eap, We1, be1r)

    hsrc1 = _sc_gather(h0, src3)
    msg1 = pl.pallas_call(
        _msg64_body,
        grid=(E_PAD // _TE,),
        in_specs=[_rows(_TE, HID), _rows(_TE, 128),
                  _full((HID, HID * HID)), _full((HID * HID, HID)),
                  _full((HID, HID))],
        out_specs=_rows(_TE, 2 * HID),
        out_shape=jax.ShapeDtypeStruct((E_PAD, 2 * HID), _f32),
    )(z, hsrc1, Vcat2, Esel, Bm)
    aggp1 = _sc_scatter64(msg1, dst3, zeros_tab64)

    h1 = pl.pallas_call(
        _combine_body,
        grid=(N_PAD // _BN5,),
        in_specs=[_rows(_BN5, 2 * HID)] * 2
        + [_rows(_BN5, 128), _full((HID, HID)), _full((1, HID))],
        out_specs=_rows(_BN5, 128),
        out_shape=jax.ShapeDtypeStruct((N_PAD, 128), _f32),
    )(aggp1[0], aggp1[1], h0, root1, b1r)

    hsrc2 = _sc_gather(h1, src3)
    msg2 = msg_call(hsrc2)
    aggp2 = _sc_scatter32(msg2, dst3, zeros_tab)

    out = pl.pallas_call(
        _combine_head_body,
        grid=(N_PAD // _BN5,),
        in_specs=[_rows(_BN5, HID)] * 2 + [_rows(_BN5, 2 * HID)] * 2
        + [_rows(_BN5, 128), _full((HID, HID)), _full((1, HID)),
           _full((HID, 1)), _full((1, 1))],
        out_specs=_rows(_BN5, 1),
        out_shape=jax.ShapeDtypeStruct((N_PAD, 1), _f32),
    )(aggp2[0], aggp2[1], aggp1[0], aggp1[1], h1, root2, b2r, Wout, boutr)

    return out[:N]


# speedup vs baseline: 1.0504x; 1.0227x over previous
import functools

import jax
import jax.numpy as jnp
from jax import lax
from jax.experimental import pallas as pl
from jax.experimental.pallas import tpu as pltpu
from jax.experimental.pallas import tpu_sc as plsc

N = 10000
E = 160000
HID = 32

NC = 2
NS = 16
NW = NC * NS
CB = 128
CH = 40
EPT = CH * CB
E_PAD = NW * EPT
N_PAD = 10240
RPS = N_PAD // NS

_f32 = jnp.float32


def _mesh():
    return plsc.VectorSubcoreMesh(core_axis_name="c", subcore_axis_name="s")


KCH = 2
NBLK = CH // KCH


def _build_gather():
    KG = 2
    NBG = CH // KG

    @functools.partial(
        pl.kernel,
        out_type=jax.ShapeDtypeStruct((E_PAD, 128), _f32),
        mesh=_mesh(),
        scratch_types=[
            pltpu.VMEM((CH, CB), jnp.int32),
            pltpu.VMEM((KG * CB, 128), _f32),
            pltpu.SemaphoreType.DMA,
        ],
    )
    def gather(table_hbm, src_hbm, out_hbm, idx_v, rows_v, sem):
        c = lax.axis_index("c")
        s = lax.axis_index("s")
        w = c * NS + s
        pltpu.sync_copy(src_hbm.at[w], idx_v)

        def body(b, carry):
            descs = [
                pltpu.async_copy(table_hbm.at[idx_v.at[b * KG + t]],
                                 rows_v.at[pl.ds(t * CB, CB)], sem)
                for t in range(KG)
            ]
            for d in descs:
                d.wait()
            pltpu.sync_copy(
                rows_v, out_hbm.at[pl.ds(w * EPT + b * (KG * CB), KG * CB)])
            return carry

        lax.fori_loop(0, NBG, body, 0)

    return gather


def _build_counts():
    @functools.partial(
        pl.kernel,
        out_type=jax.ShapeDtypeStruct((NC, N_PAD, HID), _f32),
        mesh=_mesh(),
        scratch_types=[
            pltpu.VMEM((CH, CB), jnp.int32),
            pltpu.VMEM((CB, HID), _f32),
            pltpu.VMEM_SHARED((N_PAD, HID), _f32),
            pltpu.SemaphoreType.DMA,
        ],
    )
    def counts(dst_hbm, zeros_hbm, ones_hbm, cnt_out, idx_v, ones_v, cnt_sh,
               sem):
        c = lax.axis_index("c")
        s = lax.axis_index("s")
        w = c * NS + s
        pltpu.sync_copy(dst_hbm.at[w], idx_v)
        pltpu.sync_copy(ones_hbm, ones_v)
        pltpu.sync_copy(zeros_hbm.at[pl.ds(s * RPS, RPS)],
                        cnt_sh.at[pl.ds(s * RPS, RPS)])
        plsc.subcore_barrier()

        def body(b, carry):
            descs = [
                pltpu.async_copy(ones_v, cnt_sh.at[idx_v.at[b * KCH + t]],
                                 sem, add=True)
                for t in range(KCH)
            ]
            for d in descs:
                d.wait()
            return carry

        lax.fori_loop(0, NBLK, body, 0)
        plsc.subcore_barrier()
        pltpu.sync_copy(cnt_sh.at[pl.ds(s * RPS, RPS)],
                        cnt_out.at[c, pl.ds(s * RPS, RPS)])

    return counts


def _build_scatter(W):
    @functools.partial(
        pl.kernel,
        out_type=jax.ShapeDtypeStruct((NC, N_PAD, W), _f32),
        mesh=_mesh(),
        scratch_types=[
            pltpu.VMEM((CH, CB), jnp.int32),
            pltpu.VMEM((KCH * CB, W), _f32),
            pltpu.VMEM_SHARED((N_PAD, W), _f32),
            pltpu.SemaphoreType.DMA,
        ],
    )
    def scatter(msg_hbm, dst_hbm, zeros_hbm, agg_out, idx_v, msg_v, agg_sh,
                sem):
        c = lax.axis_index("c")
        s = lax.axis_index("s")
        w = c * NS + s
        pltpu.sync_copy(dst_hbm.at[w], idx_v)
        pltpu.sync_copy(zeros_hbm.at[pl.ds(s * RPS, RPS)],
                        agg_sh.at[pl.ds(s * RPS, RPS)])
        plsc.subcore_barrier()

        def body(b, carry):
            pltpu.sync_copy(
                msg_hbm.at[pl.ds(w * EPT + b * (KCH * CB), KCH * CB)], msg_v)
            descs = [
                pltpu.async_copy(msg_v.at[pl.ds(t * CB, CB)],
                                 agg_sh.at[idx_v.at[b * KCH + t]], sem,
                                 add=True)
                for t in range(KCH)
            ]
            for d in descs:
                d.wait()
            return carry

        lax.fori_loop(0, NBLK, body, 0)
        plsc.subcore_barrier()
        pltpu.sync_copy(agg_sh.at[pl.ds(s * RPS, RPS)],
                        agg_out.at[c, pl.ds(s * RPS, RPS)])

    return scatter


def _build_counts():
    @functools.partial(
        pl.kernel,
        out_type=jax.ShapeDtypeStruct((NC, N_PAD, HID), _f32),
        mesh=_mesh(),
        scratch_types=[
            pltpu.VMEM((CH, CB), jnp.int32),
            pltpu.VMEM((CB, HID), _f32),
            pltpu.VMEM_SHARED((N_PAD, HID), _f32),
            pltpu.SemaphoreType.DMA,
        ],
    )
    def counts(dst_hbm, zeros_hbm, ones_hbm, cnt_out, idx_v, ones_v, cnt_sh,
               sem):
        c = lax.axis_index("c")
        s = lax.axis_index("s")
        w = c * NS + s
        pltpu.sync_copy(dst_hbm.at[w], idx_v)
        pltpu.sync_copy(ones_hbm, ones_v)
        pltpu.sync_copy(zeros_hbm.at[pl.ds(s * RPS, RPS)],
                        cnt_sh.at[pl.ds(s * RPS, RPS)])
        plsc.subcore_barrier()

        def body(b, carry):
            descs = [
                pltpu.async_copy(ones_v, cnt_sh.at[idx_v.at[b * KCH + t]],
                                 sem, add=True)
                for t in range(KCH)
            ]
            for d in descs:
                d.wait()
            return carry

        lax.fori_loop(0, NBLK, body, 0)
        plsc.subcore_barrier()
        pltpu.sync_copy(cnt_sh.at[pl.ds(s * RPS, RPS)],
                        cnt_out.at[c, pl.ds(s * RPS, RPS)])

    return counts


_sc_gather = _build_gather()
_sc_scatter = _build_scatter(HID)
_sc_counts = _build_counts()


_BN1 = 1024
_TE2 = 2048
_TE = 1024
_BN5 = 512


def _h0_body(x_ref, t_ref, wt1_ref, bt1_ref, wt2_ref, bt2_ref, out_ref):
    t1 = jnp.maximum(
        jnp.dot(t_ref[...], wt1_ref[...], preferred_element_type=_f32)
        + bt1_ref[...], 0.0)
    t2 = jnp.maximum(
        jnp.dot(t1, wt2_ref[...], preferred_element_type=_f32)
        + bt2_ref[...], 0.0)
    pad = jnp.zeros((x_ref.shape[0], 128 - HID), _f32)
    out_ref[...] = jnp.concatenate([x_ref[...], t2, pad], axis=1)


def _z_body(ea_ref, we1_ref, be1_ref, out_ref):
    out_ref[...] = jnp.maximum(
        jnp.dot(ea_ref[...], we1_ref[...], preferred_element_type=_f32)
        + be1_ref[...], 0.0).astype(jnp.bfloat16)


def _msg_body(z_ref, h_ref, vcat_ref, esel_ref, bm_ref, out_ref):
    z = z_ref[...]
    h = h_ref[:, :HID].astype(jnp.bfloat16)
    g2 = jnp.dot(h, vcat_ref[...],
                 preferred_element_type=_f32).astype(jnp.bfloat16)
    z128 = jnp.concatenate([z, z, z, z], axis=1)
    z1024 = jnp.concatenate([z128] * 8, axis=1)
    out_ref[...] = (
        jnp.dot(z1024 * g2, esel_ref[...], preferred_element_type=_f32)
        + jnp.dot(h, bm_ref[...], preferred_element_type=_f32))


def _combine_body(a0_ref, a1_ref, c0_ref, c1_ref, h_ref, root_ref, bias_ref,
                  out_ref):
    denom = jnp.maximum(c0_ref[...] + c1_ref[...], 1.0)
    agg = (a0_ref[...] + a1_ref[...]) / denom
    h1 = jnp.maximum(
        agg + jnp.dot(h_ref[:, :HID], root_ref[...],
                      preferred_element_type=_f32)
        + bias_ref[...], 0.0)
    pad = jnp.zeros((h1.shape[0], 128 - HID), _f32)
    out_ref[...] = jnp.concatenate([h1, pad], axis=1)


def _combine_head_body(a0_ref, a1_ref, c0_ref, c1_ref, h_ref, root_ref,
                       bias_ref, wout_ref, bout_ref, out_ref):
    denom = jnp.maximum(c0_ref[...] + c1_ref[...], 1.0)
    agg = (a0_ref[...] + a1_ref[...]) / denom
    h2 = jnp.maximum(
        agg + jnp.dot(h_ref[:, :HID], root_ref[...],
                      preferred_element_type=_f32)
        + bias_ref[...], 0.0)
    logits = jnp.dot(h2, wout_ref[...], preferred_element_type=_f32) \
        + bout_ref[...]
    out_ref[...] = jax.nn.sigmoid(logits)


def _full(shape):
    return pl.BlockSpec(shape, lambda i: (0,) * len(shape))


def _rows(bn, w):
    return pl.BlockSpec((bn, w), lambda i: (i, 0))


def kernel(x, topo, edge_attr, edge_index, Wt1, bt1, Wt2, bt2, We1, be1,
           We2, be2, root1, bias1, root2, bias2, Wout, bout):
    node_in = x.shape[1]
    topo_in = topo.shape[1]
    tproj = Wt1.shape[1]
    edge_in = edge_attr.shape[1]

    xp = jnp.pad(x, ((0, N_PAD - N), (0, 0)))
    tp = jnp.pad(topo, ((0, N_PAD - N), (0, 0)))
    eap = jnp.pad(edge_attr, ((0, E_PAD - E), (0, 0)))
    src3 = jnp.pad(edge_index[0], (0, E_PAD - E)).reshape(NW, CH, CB)
    dst3 = jnp.pad(edge_index[1], (0, E_PAD - E),
                   constant_values=N).reshape(NW, CH, CB)
    Vcat2 = (We2.reshape(HID, HID, HID).transpose(1, 2, 0)
             .reshape(HID, HID * HID).astype(jnp.bfloat16))
    Esel = jnp.kron(jnp.eye(HID, dtype=_f32),
                    jnp.ones((HID, 1), _f32)).astype(jnp.bfloat16)
    Bm = be2.reshape(HID, HID)
    zeros_tab = jnp.zeros((N_PAD, HID), _f32)
    ones_blk = jnp.ones((CB, HID), _f32)
    bt1r = bt1.reshape(1, tproj)
    bt2r = bt2.reshape(1, tproj)
    be1r = be1.reshape(1, HID)
    b1r = bias1.reshape(1, HID)
    b2r = bias2.reshape(1, HID)
    boutr = bout.reshape(1, 1)

    h0 = pl.pallas_call(
        _h0_body,
        grid=(N_PAD // _BN1,),
        in_specs=[_rows(_BN1, node_in), _rows(_BN1, topo_in),
                  _full((topo_in, tproj)), _full((1, tproj)),
                  _full((tproj, tproj)), _full((1, tproj))],
        out_specs=_rows(_BN1, 128),
        out_shape=jax.ShapeDtypeStruct((N_PAD, 128), _f32),
    )(xp, tp, Wt1, bt1r, Wt2, bt2r)

    def msg_call(hsrc):
        return pl.pallas_call(
            _msg_body,
            grid=(E_PAD // _TE,),
            in_specs=[_rows(_TE, HID), _rows(_TE, 128),
                      _full((HID, HID * HID)), _full((HID * HID, HID)),
                      _full((HID, HID))],
            out_specs=_rows(_TE, HID),
            out_shape=jax.ShapeDtypeStruct((E_PAD, HID), _f32),
        )(z, hsrc, Vcat2, Esel, Bm)

    z = pl.pallas_call(
        _z_body,
        grid=(E_PAD // _TE2,),
        in_specs=[_rows(_TE2, edge_in), _full((edge_in, HID)),
                  _full((1, HID))],
        out_specs=_rows(_TE2, HID),
        out_shape=jax.ShapeDtypeStruct((E_PAD, HID), jnp.bfloat16),
    )(eap, We1, be1r)

    cntp = _sc_counts(dst3, zeros_tab, ones_blk)
    hsrc1 = _sc_gather(h0, src3)
    msg1 = msg_call(hsrc1)
    aggp1 = _sc_scatter(msg1, dst3, zeros_tab)

    h1 = pl.pallas_call(
        _combine_body,
        grid=(N_PAD // _BN5,),
        in_specs=[_rows(_BN5, HID)] * 4
        + [_rows(_BN5, 128), _full((HID, HID)), _full((1, HID))],
        out_specs=_rows(_BN5, 128),
        out_shape=jax.ShapeDtypeStruct((N_PAD, 128), _f32),
    )(aggp1[0], aggp1[1], cntp[0], cntp[1], h0, root1, b1r)

    hsrc2 = _sc_gather(h1, src3)
    msg2 = msg_call(hsrc2)
    aggp2 = _sc_scatter(msg2, dst3, zeros_tab)

    out = pl.pallas_call(
        _combine_head_body,
        grid=(N_PAD // _BN5,),
        in_specs=[_rows(_BN5, HID)] * 4
        + [_rows(_BN5, 128), _full((HID, HID)), _full((1, HID)),
           _full((HID, 1)), _full((1, 1))],
        out_specs=_rows(_BN5, 1),
        out_shape=jax.ShapeDtypeStruct((N_PAD, 1), _f32),
    )(aggp2[0], aggp2[1], cntp[0], cntp[1], h1, root2, b2r, Wout, boutr)

    return out[:N]
